# baseline (speedup 1.0000x reference)
import functools

import jax
import jax.numpy as jnp
from jax import lax
from jax.experimental import pallas as pl
from jax.experimental.pallas import tpu as pltpu, tpu_sc as plsc

N = 10000
E = 320000
DIN = 128
DH = 256
DOUT = 64
EPS = 1e-5

HF = 128
CH = 128
CPT = 160
IBLK = 16
EP = 16 * CPT * CH
NDUMP = N + 8
DEG_CHUNKS = EP // 32 // CH
RPT = 624
TAIL_BASE = 16 * RPT
TAIL = N - TAIL_BASE

def _zero_rows(ref, nrows, ncol16):
    z16 = jnp.zeros((16,), jnp.float32)

    def body(i, carry):
        for k in range(ncol16):
            ref[i, k * 16:(k + 1) * 16] = z16
        return carry

    lax.fori_loop(0, nrows, body, 0)


def _zero_acc_slice(zbuf, acc, s):
    rbase = s * RPT
    for k in range(RPT // CH):
        pltpu.sync_copy(zbuf, acc.at[pl.ds(rbase + k * CH, CH)])
    rem = RPT - (RPT // CH) * CH
    pltpu.sync_copy(zbuf.at[pl.ds(0, rem)],
                    acc.at[pl.ds(rbase + (RPT // CH) * CH, rem)])

    @pl.when(s == 0)
    def _():
        pltpu.sync_copy(zbuf.at[pl.ds(0, NDUMP - TAIL_BASE)],
                        acc.at[pl.ds(TAIL_BASE, NDUMP - TAIL_BASE)])


def _copy_out_slice(acc, out_hbm, c, s):
    rbase = s * RPT
    pltpu.sync_copy(acc.at[pl.ds(rbase, RPT)],
                    out_hbm.at[pl.ds(c * N + rbase, RPT)])

    @pl.when(s == 0)
    def _():
        pltpu.sync_copy(acc.at[pl.ds(TAIL_BASE, TAIL)],
                        out_hbm.at[pl.ds(c * N + TAIL_BASE, TAIL)])


@functools.cache
def _make_deg_kernel():
    mesh = plsc.VectorSubcoreMesh(core_axis_name="c", subcore_axis_name="s")
    return functools.partial(
        pl.kernel,
        mesh=mesh,
        out_type=jax.ShapeDtypeStruct((2 * N, 16), jnp.float32),
        scratch_types=[
            pltpu.VMEM((DEG_CHUNKS, CH), jnp.int32),
            pltpu.VMEM((CH, 16), jnp.float32),
            pltpu.VMEM_SHARED((NDUMP, 16), jnp.float32),
        ],
    )(_deg_body)


def _deg_body(dstr_hbm, out_hbm, didx_all, buf, acc):
    c = lax.axis_index("c")
    s = lax.axis_index("s")

    _zero_rows(buf, CH, 1)
    _zero_acc_slice(buf, acc, s)

    w = c * 16 + s
    pltpu.sync_copy(dstr_hbm.at[pl.ds(w * DEG_CHUNKS, DEG_CHUNKS)], didx_all)

    o16 = jnp.ones((16,), jnp.float32)

    def ones_body(i, carry):
        buf[i, 0:16] = o16
        return carry

    lax.fori_loop(0, CH, ones_body, 0)
    plsc.subcore_barrier()

    def chunk(j, carry):
        pltpu.sync_copy(buf, acc.at[didx_all.at[j]], add=True)
        return carry

    lax.fori_loop(0, DEG_CHUNKS, chunk, 0)
    plsc.subcore_barrier()

    _copy_out_slice(acc, out_hbm, c, s)


@functools.cache
def _make_scatter_kernel():
    mesh = plsc.VectorSubcoreMesh(core_axis_name="c", subcore_axis_name="s")
    return functools.partial(
        pl.kernel,
        mesh=mesh,
        out_type=jax.ShapeDtypeStruct((2 * N, HF), jnp.float32),
        scratch_types=[
            pltpu.VMEM((2 * IBLK, CH), jnp.int32),
            pltpu.VMEM((CH, HF), jnp.float32),
            pltpu.VMEM((CH, HF), jnp.float32),
            pltpu.VMEM_SHARED((NDUMP, HF), jnp.float32),
            pltpu.SemaphoreType.DMA,
            pltpu.SemaphoreType.DMA,
        ],
    )(_scatter_body)


def _scatter_body(y2_hbm, idxr_hbm, out_hbm, idx_blk, rows0, rows1, acc, sem0,
                  sem1):
    c = lax.axis_index("c")
    s = lax.axis_index("s")

    _zero_rows(rows0, CH, HF // 16)
    _zero_acc_slice(rows0, acc, s)
    plsc.subcore_barrier()

    tile_row0 = (c * (EP // CH) + s * CPT) * 2

    def gather_start(e, rows, sem):
        return pltpu.async_copy(y2_hbm.at[idx_blk.at[2 * e]], rows, sem)

    def gather_wait(e, rows, sem):
        pltpu.make_async_copy(y2_hbm.at[idx_blk.at[2 * e]], rows, sem).wait()

    def scat(e, rows):
        pltpu.sync_copy(rows, acc.at[idx_blk.at[2 * e + 1]], add=True)

    def block(blk, carry):
        pltpu.sync_copy(
            idxr_hbm.at[pl.ds(tile_row0 + blk * (2 * IBLK), 2 * IBLK)],
            idx_blk)
        gather_start(0, rows0, sem0)

        def pair(j, carry):
            e0 = 2 * j
            e1 = e0 + 1
            e2 = jnp.minimum(e0 + 2, IBLK - 1)
            gather_start(e1, rows1, sem1)
            gather_wait(e0, rows0, sem0)
            scat(e0, rows0)
            gather_start(e2, rows0, sem0)
            gather_wait(e1, rows1, sem1)
            scat(e1, rows1)
            return carry

        lax.fori_loop(0, IBLK // 2, pair, 0)
        gather_wait(IBLK - 1, rows0, sem0)
        return carry

    lax.fori_loop(0, CPT // IBLK, block, 0)
    plsc.subcore_barrier()

    _copy_out_slice(acc, out_hbm, c, s)


BLK = 1000
NBLK = N // BLK


def _prep_body(x_ref, w_ref, dega_ref, degb_ref, y2_ref, dinv_ref):
    deg = dega_ref[:, 0:1] + degb_ref[:, 0:1] + 1.0
    dinv = lax.rsqrt(deg)
    xw = jnp.dot(x_ref[...], w_ref[...], preferred_element_type=jnp.float32)
    y = xw * dinv
    y2_ref[0] = y[:, :HF]
    y2_ref[1] = y[:, HF:]
    dinv_ref[...] = jnp.broadcast_to(dinv, (BLK, HF))


def _prep(x, w0, degp):
    degf = degp.reshape(2 * N, 16)
    return pl.pallas_call(
        _prep_body,
        grid=(NBLK,),
        in_specs=[
            pl.BlockSpec((BLK, DIN), lambda i: (i, 0)),
            pl.BlockSpec((DIN, DH), lambda i: (0, 0)),
            pl.BlockSpec((BLK, 16), lambda i: (i, 0)),
            pl.BlockSpec((BLK, 16), lambda i: (i + NBLK, 0)),
        ],
        out_specs=[
            pl.BlockSpec((2, BLK, HF), lambda i: (0, i, 0)),
            pl.BlockSpec((BLK, HF), lambda i: (i, 0)),
        ],
        out_shape=[
            jax.ShapeDtypeStruct((2, N, HF), jnp.float32),
            jax.ShapeDtypeStruct((N, HF), jnp.float32),
        ],
    )(x, w0, degf, degf)


def _stats_body(s_ref, y_ref, dinv_ref, b_ref, z_ref, st_ref):
    i = pl.program_id(0)
    dinv = dinv_ref[...]
    z0 = dinv * (s_ref[0] + y_ref[0]) + b_ref[0:1, :HF]
    z1 = dinv * (s_ref[1] + y_ref[1]) + b_ref[0:1, HF:]
    z_ref[0] = z0
    z_ref[1] = z1

    @pl.when(i == 0)
    def _():
        st_ref[...] = jnp.zeros((2, 8, HF), jnp.float32)

    st_ref[0, 0:1, :] += jnp.sum(z0, axis=0, keepdims=True)
    st_ref[0, 1:2, :] += jnp.sum(z0 * z0, axis=0, keepdims=True)
    st_ref[1, 0:1, :] += jnp.sum(z1, axis=0, keepdims=True)
    st_ref[1, 1:2, :] += jnp.sum(z1 * z1, axis=0, keepdims=True)


def _stats(s2, y2, dinv, b):
    bpad = jnp.broadcast_to(b.reshape(1, DH), (8, DH))
    return pl.pallas_call(
        _stats_body,
        grid=(NBLK,),
        in_specs=[
            pl.BlockSpec((2, BLK, HF), lambda i: (0, i, 0)),
            pl.BlockSpec((2, BLK, HF), lambda i: (0, i, 0)),
            pl.BlockSpec((BLK, HF), lambda i: (i, 0)),
            pl.BlockSpec((8, DH), lambda i: (0, 0)),
        ],
        out_specs=[
            pl.BlockSpec((2, BLK, HF), lambda i: (0, i, 0)),
            pl.BlockSpec((2, 8, HF), lambda i: (0, 0, 0)),
        ],
        out_shape=[
            jax.ShapeDtypeStruct((2, N, HF), jnp.float32),
            jax.ShapeDtypeStruct((2, 8, HF), jnp.float32),
        ],
    )(s2, y2, dinv, bpad)


def _bn_relu_halves(z_ref, st_ref, g_ref, be_ref):
    hs = []
    for k in range(2):
        m = st_ref[k, 0:1, :] * (1.0 / N)
        ex2 = st_ref[k, 1:2, :] * (1.0 / N)
        inv = lax.rsqrt(ex2 - m * m + EPS)
        h = (z_ref[k] - m) * inv * g_ref[0:1, k * HF:(k + 1) * HF]
        h = h + be_ref[0:1, k * HF:(k + 1) * HF]
        hs.append(jnp.maximum(h, 0.0))
    return jnp.concatenate(hs, axis=1)


def _mid_body(z_ref, st_ref, g_ref, be_ref, w_ref, dinv_ref, y2_ref):
    h = _bn_relu_halves(z_ref, st_ref, g_ref, be_ref)
    y = jnp.dot(h, w_ref[...], preferred_element_type=jnp.float32)
    y = y * dinv_ref[...][:, 0:1]
    y2_ref[0] = y[:, :HF]
    y2_ref[1] = y[:, HF:]


def _mid(z2, st, g, be, w, dinv):
    gpad = jnp.broadcast_to(g.reshape(1, DH), (8, DH))
    bepad = jnp.broadcast_to(be.reshape(1, DH), (8, DH))
    return pl.pallas_call(
        _mid_body,
        grid=(NBLK,),
        in_specs=[
            pl.BlockSpec((2, BLK, HF), lambda i: (0, i, 0)),
            pl.BlockSpec((2, 8, HF), lambda i: (0, 0, 0)),
            pl.BlockSpec((8, DH), lambda i: (0, 0)),
            pl.BlockSpec((8, DH), lambda i: (0, 0)),
            pl.BlockSpec((DH, DH), lambda i: (0, 0)),
            pl.BlockSpec((BLK, HF), lambda i: (i, 0)),
        ],
        out_specs=pl.BlockSpec((2, BLK, HF), lambda i: (0, i, 0)),
        out_shape=jax.ShapeDtypeStruct((2, N, HF), jnp.float32),
    )(z2, st, gpad, bepad, w, dinv)


def _head_body(z_ref, st_ref, g_ref, be_ref, w_ref, bo_ref, o_ref):
    h = _bn_relu_halves(z_ref, st_ref, g_ref, be_ref)
    o_ref[...] = jnp.dot(h, w_ref[...],
                         preferred_element_type=jnp.float32) + bo_ref[0:1, :]


def _head(z2, st, g, be, wo, bo):
    gpad = jnp.broadcast_to(g.reshape(1, DH), (8, DH))
    bepad = jnp.broadcast_to(be.reshape(1, DH), (8, DH))
    bopad = jnp.broadcast_to(bo.reshape(1, DOUT), (8, DOUT))
    return pl.pallas_call(
        _head_body,
        grid=(NBLK,),
        in_specs=[
            pl.BlockSpec((2, BLK, HF), lambda i: (0, i, 0)),
            pl.BlockSpec((2, 8, HF), lambda i: (0, 0, 0)),
            pl.BlockSpec((8, DH), lambda i: (0, 0)),
            pl.BlockSpec((8, DH), lambda i: (0, 0)),
            pl.BlockSpec((DH, DOUT), lambda i: (0, 0)),
            pl.BlockSpec((8, DOUT), lambda i: (0, 0)),
        ],
        out_specs=pl.BlockSpec((BLK, DOUT), lambda i: (i, 0)),
        out_shape=jax.ShapeDtypeStruct((N, DOUT), jnp.float32),
    )(z2, st, gpad, bepad, wo, bopad)


def kernel(x, edge_index, W0, b0, g0, be0, W1, b1, g1, be1, W2, b2, g2, be2,
           Wo, bo):
    src = edge_index[0]
    dst = edge_index[1]
    srcp = jnp.concatenate([src, jnp.zeros((EP - E,), jnp.int32)])
    dstp = jnp.concatenate([dst, jnp.full((EP - E,), N, jnp.int32)])
    src2r = jnp.stack([srcp, srcp + N]).reshape(2, EP // CH, CH)
    dstr = jnp.broadcast_to(dstp.reshape(1, EP // CH, CH), (2, EP // CH, CH))
    idxr = jnp.stack([src2r, dstr], axis=2).reshape(4 * EP // CH, CH)

    degp = _make_deg_kernel()(dstp.reshape(EP // CH, CH))
    y2, dinv = _prep(x, W0, degp)

    def agg(y2):
        return _make_scatter_kernel()(y2.reshape(2 * N, HF),
                                      idxr).reshape(2, N, HF)

    z2, st = _stats(agg(y2), y2, dinv, b0)
    y2 = _mid(z2, st, g0, be0, W1, dinv)
    z2, st = _stats(agg(y2), y2, dinv, b1)
    y2 = _mid(z2, st, g1, be1, W2, dinv)
    z2, st = _stats(agg(y2), y2, dinv, b2)
    return _head(z2, st, g2, be2, Wo, bo)


# speedup vs baseline: 1.0550x; 1.0550x over previous
import functools

import jax
import jax.numpy as jnp
from jax import lax
from jax.experimental import pallas as pl
from jax.experimental.pallas import tpu as pltpu, tpu_sc as plsc

N = 10000
E = 320000
DIN = 128
DH = 256
DOUT = 64
EPS = 1e-5

HF = 128
CH = 80
EDGES_PER_TILE = E // 16
CHUNKS_PER_TILE = EDGES_PER_TILE // CH
DCH = 128
EP = 32 * 80 * DCH
DEG_CHUNKS = EP // 32 // DCH
NDUMP = N + 8
RPT = 624
TAIL_BASE = 16 * RPT
TAIL = N - TAIL_BASE


def _zero_rows(ref, nrows, ncol16):
    z16 = jnp.zeros((16,), jnp.float32)

    def body(i, carry):
        for k in range(ncol16):
            ref[i, k * 16:(k + 1) * 16] = z16
        return carry

    lax.fori_loop(0, nrows, body, 0)


def _zero_acc_slice(zbuf, acc, s, zrows):
    rbase = s * RPT
    for k in range(RPT // zrows):
        pltpu.sync_copy(zbuf, acc.at[pl.ds(rbase + k * zrows, zrows)])
    rem = RPT - (RPT // zrows) * zrows
    if rem:
        pltpu.sync_copy(zbuf.at[pl.ds(0, rem)],
                        acc.at[pl.ds(rbase + (RPT // zrows) * zrows, rem)])

    @pl.when(s == 0)
    def _():
        pltpu.sync_copy(zbuf.at[pl.ds(0, NDUMP - TAIL_BASE)],
                        acc.at[pl.ds(TAIL_BASE, NDUMP - TAIL_BASE)])


def _copy_out_slice(acc, out_hbm, base_row, s):
    rbase = s * RPT
    pltpu.sync_copy(acc.at[pl.ds(rbase, RPT)],
                    out_hbm.at[pl.ds(base_row + rbase, RPT)])

    @pl.when(s == 0)
    def _():
        pltpu.sync_copy(acc.at[pl.ds(TAIL_BASE, TAIL)],
                        out_hbm.at[pl.ds(base_row + TAIL_BASE, TAIL)])


@functools.cache
def _make_deg_kernel():
    mesh = plsc.VectorSubcoreMesh(core_axis_name="c", subcore_axis_name="s")
    return functools.partial(
        pl.kernel,
        mesh=mesh,
        out_type=jax.ShapeDtypeStruct((2 * N, 16), jnp.float32),
        scratch_types=[
            pltpu.VMEM((DEG_CHUNKS, DCH), jnp.int32),
            pltpu.VMEM((DCH, 16), jnp.float32),
            pltpu.VMEM_SHARED((NDUMP, 16), jnp.float32),
        ],
    )(_deg_body)


def _deg_body(dstr_hbm, out_hbm, didx_all, buf, acc):
    c = lax.axis_index("c")
    s = lax.axis_index("s")

    _zero_rows(buf, DCH, 1)
    _zero_acc_slice(buf, acc, s, DCH)

    w = c * 16 + s
    pltpu.sync_copy(dstr_hbm.at[pl.ds(w * DEG_CHUNKS, DEG_CHUNKS)], didx_all)

    o16 = jnp.ones((16,), jnp.float32)

    def ones_body(i, carry):
        buf[i, 0:16] = o16
        return carry

    lax.fori_loop(0, DCH, ones_body, 0)
    plsc.subcore_barrier()

    def chunk(j, carry):
        pltpu.sync_copy(buf, acc.at[didx_all.at[j]], add=True)
        return carry

    lax.fori_loop(0, DEG_CHUNKS, chunk, 0)
    plsc.subcore_barrier()

    _copy_out_slice(acc, out_hbm, c * N, s)


@functools.cache
def _make_scatter_kernel():
    mesh = plsc.VectorSubcoreMesh(core_axis_name="c", subcore_axis_name="s")
    return functools.partial(
        pl.kernel,
        mesh=mesh,
        out_type=jax.ShapeDtypeStruct((2 * N, HF), jnp.float32),
        scratch_types=[
            pltpu.VMEM((CH,), jnp.int32),
            pltpu.VMEM((CH,), jnp.int32),
            pltpu.VMEM((CH, HF), jnp.float32),
            pltpu.VMEM_SHARED((NDUMP, HF), jnp.float32),
            pltpu.SemaphoreType.DMA,
        ],
    )(_scatter_body)


def _scatter_body(y2_hbm, src2_hbm, dst_hbm, out_hbm, sidx, didx, rows, acc,
                  sem):
    c = lax.axis_index("c")
    s = lax.axis_index("s")

    _zero_rows(rows, CH, HF // 16)
    _zero_acc_slice(rows, acc, s, CH)
    plsc.subcore_barrier()

    ebase = s * EDGES_PER_TILE
    sbase = c * E + ebase

    def chunk(j, carry):
        off = j * CH
        pltpu.sync_copy(src2_hbm.at[pl.ds(sbase + off, CH)], sidx)
        pltpu.sync_copy(dst_hbm.at[pl.ds(ebase + off, CH)], didx)
        pltpu.async_copy(y2_hbm.at[sidx], rows, sem).wait()
        pltpu.sync_copy(rows, acc.at[didx], add=True)
        return carry

    lax.fori_loop(0, CHUNKS_PER_TILE, chunk, 0)
    plsc.subcore_barrier()

    _copy_out_slice(acc, out_hbm, c * N, s)


BLK = 1000
NBLK = N // BLK


def _prep_body(x_ref, w_ref, dega_ref, degb_ref, y2_ref, dinv_ref):
    deg = dega_ref[:, 0:1] + degb_ref[:, 0:1] + 1.0
    dinv = lax.rsqrt(deg)
    xw = jnp.dot(x_ref[...], w_ref[...], preferred_element_type=jnp.float32)
    y = xw * dinv
    y2_ref[0] = y[:, :HF]
    y2_ref[1] = y[:, HF:]
    dinv_ref[...] = jnp.broadcast_to(dinv, (BLK, HF))


def _prep(x, w0, degp):
    degf = degp.reshape(2 * N, 16)
    return pl.pallas_call(
        _prep_body,
        grid=(NBLK,),
        in_specs=[
            pl.BlockSpec((BLK, DIN), lambda i: (i, 0)),
            pl.BlockSpec((DIN, DH), lambda i: (0, 0)),
            pl.BlockSpec((BLK, 16), lambda i: (i, 0)),
            pl.BlockSpec((BLK, 16), lambda i: (i + NBLK, 0)),
        ],
        out_specs=[
            pl.BlockSpec((2, BLK, HF), lambda i: (0, i, 0)),
            pl.BlockSpec((BLK, HF), lambda i: (i, 0)),
        ],
        out_shape=[
            jax.ShapeDtypeStruct((2, N, HF), jnp.float32),
            jax.ShapeDtypeStruct((N, HF), jnp.float32),
        ],
    )(x, w0, degf, degf)


def _stats_body(s_ref, y_ref, dinv_ref, b_ref, z_ref, st_ref):
    i = pl.program_id(0)
    dinv = dinv_ref[...]
    z0 = dinv * (s_ref[0] + y_ref[0]) + b_ref[0:1, :HF]
    z1 = dinv * (s_ref[1] + y_ref[1]) + b_ref[0:1, HF:]
    z_ref[0] = z0
    z_ref[1] = z1

    @pl.when(i == 0)
    def _():
        st_ref[...] = jnp.zeros((2, 8, HF), jnp.float32)

    st_ref[0, 0:1, :] += jnp.sum(z0, axis=0, keepdims=True)
    st_ref[0, 1:2, :] += jnp.sum(z0 * z0, axis=0, keepdims=True)
    st_ref[1, 0:1, :] += jnp.sum(z1, axis=0, keepdims=True)
    st_ref[1, 1:2, :] += jnp.sum(z1 * z1, axis=0, keepdims=True)


def _stats(s2, y2, dinv, b):
    bpad = jnp.broadcast_to(b.reshape(1, DH), (8, DH))
    return pl.pallas_call(
        _stats_body,
        grid=(NBLK,),
        in_specs=[
            pl.BlockSpec((2, BLK, HF), lambda i: (0, i, 0)),
            pl.BlockSpec((2, BLK, HF), lambda i: (0, i, 0)),
            pl.BlockSpec((BLK, HF), lambda i: (i, 0)),
            pl.BlockSpec((8, DH), lambda i: (0, 0)),
        ],
        out_specs=[
            pl.BlockSpec((2, BLK, HF), lambda i: (0, i, 0)),
            pl.BlockSpec((2, 8, HF), lambda i: (0, 0, 0)),
        ],
        out_shape=[
            jax.ShapeDtypeStruct((2, N, HF), jnp.float32),
            jax.ShapeDtypeStruct((2, 8, HF), jnp.float32),
        ],
    )(s2, y2, dinv, bpad)


def _bn_relu_halves(z_ref, st_ref, g_ref, be_ref):
    hs = []
    for k in range(2):
        m = st_ref[k, 0:1, :] * (1.0 / N)
        ex2 = st_ref[k, 1:2, :] * (1.0 / N)
        inv = lax.rsqrt(ex2 - m * m + EPS)
        h = (z_ref[k] - m) * inv * g_ref[0:1, k * HF:(k + 1) * HF]
        h = h + be_ref[0:1, k * HF:(k + 1) * HF]
        hs.append(jnp.maximum(h, 0.0))
    return jnp.concatenate(hs, axis=1)


def _mid_body(z_ref, st_ref, g_ref, be_ref, w_ref, dinv_ref, y2_ref):
    h = _bn_relu_halves(z_ref, st_ref, g_ref, be_ref)
    y = jnp.dot(h, w_ref[...], preferred_element_type=jnp.float32)
    y = y * dinv_ref[...][:, 0:1]
    y2_ref[0] = y[:, :HF]
    y2_ref[1] = y[:, HF:]


def _mid(z2, st, g, be, w, dinv):
    gpad = jnp.broadcast_to(g.reshape(1, DH), (8, DH))
    bepad = jnp.broadcast_to(be.reshape(1, DH), (8, DH))
    return pl.pallas_call(
        _mid_body,
        grid=(NBLK,),
        in_specs=[
            pl.BlockSpec((2, BLK, HF), lambda i: (0, i, 0)),
            pl.BlockSpec((2, 8, HF), lambda i: (0, 0, 0)),
            pl.BlockSpec((8, DH), lambda i: (0, 0)),
            pl.BlockSpec((8, DH), lambda i: (0, 0)),
            pl.BlockSpec((DH, DH), lambda i: (0, 0)),
            pl.BlockSpec((BLK, HF), lambda i: (i, 0)),
        ],
        out_specs=pl.BlockSpec((2, BLK, HF), lambda i: (0, i, 0)),
        out_shape=jax.ShapeDtypeStruct((2, N, HF), jnp.float32),
    )(z2, st, gpad, bepad, w, dinv)


def _head_body(z_ref, st_ref, g_ref, be_ref, w_ref, bo_ref, o_ref):
    h = _bn_relu_halves(z_ref, st_ref, g_ref, be_ref)
    o_ref[...] = jnp.dot(h, w_ref[...],
                         preferred_element_type=jnp.float32) + bo_ref[0:1, :]


def _head(z2, st, g, be, wo, bo):
    gpad = jnp.broadcast_to(g.reshape(1, DH), (8, DH))
    bepad = jnp.broadcast_to(be.reshape(1, DH), (8, DH))
    bopad = jnp.broadcast_to(bo.reshape(1, DOUT), (8, DOUT))
    return pl.pallas_call(
        _head_body,
        grid=(NBLK,),
        in_specs=[
            pl.BlockSpec((2, BLK, HF), lambda i: (0, i, 0)),
            pl.BlockSpec((2, 8, HF), lambda i: (0, 0, 0)),
            pl.BlockSpec((8, DH), lambda i: (0, 0)),
            pl.BlockSpec((8, DH), lambda i: (0, 0)),
            pl.BlockSpec((DH, DOUT), lambda i: (0, 0)),
            pl.BlockSpec((8, DOUT), lambda i: (0, 0)),
        ],
        out_specs=pl.BlockSpec((BLK, DOUT), lambda i: (i, 0)),
        out_shape=jax.ShapeDtypeStruct((N, DOUT), jnp.float32),
    )(z2, st, gpad, bepad, wo, bopad)


def kernel(x, edge_index, W0, b0, g0, be0, W1, b1, g1, be1, W2, b2, g2, be2,
           Wo, bo):
    src = edge_index[0]
    dst = edge_index[1]
    src2 = jnp.concatenate([src, src + N])
    dstp = jnp.concatenate([dst, jnp.full((EP - E,), N, jnp.int32)])

    degp = _make_deg_kernel()(dstp.reshape(EP // DCH, DCH))
    y2, dinv = _prep(x, W0, degp)

    def agg(y2):
        return _make_scatter_kernel()(y2.reshape(2 * N, HF), src2,
                                      dst).reshape(2, N, HF)

    z2, st = _stats(agg(y2), y2, dinv, b0)
    y2 = _mid(z2, st, g0, be0, W1, dinv)
    z2, st = _stats(agg(y2), y2, dinv, b1)
    y2 = _mid(z2, st, g1, be1, W2, dinv)
    z2, st = _stats(agg(y2), y2, dinv, b2)
    return _head(z2, st, g2, be2, Wo, bo)
